# Initial kernel scaffold; baseline (speedup 1.0000x reference)
import jax
import jax.numpy as jnp
from jax.experimental import pallas as pl


def kernel(x, batch, edge_index, W1, b1, W2, b2, W3, b3, fc_w, fc_b, lin1_w, lin1_b, lin2_w, lin2_b):
    raise NotImplementedError("write your pallas kernel here")


# speedup vs baseline: 39.3094x; 39.3094x over previous
import functools
import jax
import jax.numpy as jnp
from jax import lax
from jax.experimental import pallas as pl
from jax.experimental.pallas import tpu as pltpu, tpu_sc as plsc

N = 15135
NP = 15360
NPH = NP // 2
E = 484320
EP = 491520
EPR = EP // 128
F_IN = 128
H = 64
HP = 128
HFC = 256
NC = 10
BS = 2

NB = 512
RB = NP // NB
CHUNK = 512
TROWS = NPH // 16

_SC_MESH = plsc.VectorSubcoreMesh(core_axis_name="c", subcore_axis_name="s")


@functools.partial(
    pl.kernel,
    out_type=jax.ShapeDtypeStruct((NP, HP), jnp.float32),
    mesh=_SC_MESH,
    scratch_types=[
        pltpu.VMEM((4, 128), jnp.int32),
        pltpu.VMEM((128,), jnp.int32),
        pltpu.VMEM((128,), jnp.int32),
        pltpu.VMEM((128,), jnp.int32),
        pltpu.VMEM((128,), jnp.int32),
        pltpu.VMEM((CHUNK, HP), jnp.float32),
        pltpu.VMEM_SHARED((NPH + 8, HP), jnp.float32),
        pltpu.SemaphoreType.DMA,
    ],
)
def _sc_scatter(xs, src2, dst2, out, sidx, didx0, didx1, didx2, didx3,
                rows, acc, sem):
    c = lax.axis_index("c")
    t = lax.axis_index("s")
    didx = [didx0, didx1, didx2, didx3]
    zero16 = jnp.zeros((16,), jnp.float32)
    nph16 = jnp.full((16,), NPH, jnp.int32)

    def zfill(i, carry):
        for k in range(8):
            rows[i, pl.ds(k * 16, 16)] = zero16
        return carry

    lax.fori_loop(0, TROWS, zfill, 0)

    base = t * TROWS
    pltpu.sync_copy(rows.at[pl.ds(0, TROWS)], acc.at[pl.ds(base, TROWS)])
    plsc.subcore_barrier()

    off = c * NPH

    def body(i, carry):
        rowbase = t * 240 + i * 4
        pltpu.sync_copy(src2.at[pl.ds(rowbase, 4)], sidx)
        for j in range(4):
            pltpu.sync_copy(dst2.at[rowbase + j], didx[j])
        handles = [
            pltpu.async_copy(xs.at[sidx.at[j]],
                             rows.at[pl.ds(j * 128, 128)], sem)
            for j in range(4)
        ]
        for h in handles:
            h.wait()
        for j in range(4):
            for k in range(8):
                sl = pl.ds(k * 16, 16)
                v = didx[j][sl] - off
                vu = plsc.bitcast(v, jnp.uint32)
                didx[j][sl] = jnp.where(vu < NPH, v, nph16)
        for j in range(4):
            pltpu.sync_copy(rows.at[pl.ds(j * 128, 128)],
                            acc.at[didx[j]], add=True)
        return carry

    lax.fori_loop(0, 60, body, 0)
    plsc.subcore_barrier()

    pltpu.sync_copy(acc.at[pl.ds(base, TROWS)], rows.at[pl.ds(0, TROWS)])
    pltpu.sync_copy(rows.at[pl.ds(0, TROWS)], out.at[pl.ds(off + base, TROWS)])


def _t1_body(xp_ref, s_ref, w1_ref, xs1_ref, dinv_ref):
    dinvb = lax.rsqrt(s_ref[...] + 1.0)
    xw = jnp.dot(xp_ref[...], w1_ref[...], preferred_element_type=jnp.float32)
    xs1_ref[...] = dinvb * xw
    dinv_ref[...] = dinvb


def _t1(xp, s0, w1b):
    return pl.pallas_call(
        _t1_body,
        grid=(RB,),
        in_specs=[
            pl.BlockSpec((NB, 2 * F_IN), lambda r: (r, 0)),
            pl.BlockSpec((NB, HP), lambda r: (r, 0)),
            pl.BlockSpec((2 * F_IN, HP), lambda r: (0, 0)),
        ],
        out_specs=[
            pl.BlockSpec((NB, HP), lambda r: (r, 0)),
            pl.BlockSpec((NB, HP), lambda r: (r, 0)),
        ],
        out_shape=[
            jax.ShapeDtypeStruct((NP, HP), jnp.float32),
            jax.ShapeDtypeStruct((NP, HP), jnp.float32),
        ],
    )(xp, s0, w1b)


def _t2_body(xs_ref, s_ref, dinv_ref, b_ref, w_ref, xl_ref, xsn_ref):
    xl = jnp.maximum(dinv_ref[...] * (xs_ref[...] + s_ref[...]) + b_ref[...], 0.0)
    xl_ref[...] = xl
    xw = jnp.dot(xl, w_ref[...], preferred_element_type=jnp.float32)
    xsn_ref[...] = dinv_ref[...] * xw


def _t2(xs, s, dinvb, b, w):
    return pl.pallas_call(
        _t2_body,
        grid=(RB,),
        in_specs=[
            pl.BlockSpec((NB, HP), lambda r: (r, 0)),
            pl.BlockSpec((NB, HP), lambda r: (r, 0)),
            pl.BlockSpec((NB, HP), lambda r: (r, 0)),
            pl.BlockSpec((1, HP), lambda r: (0, 0)),
            pl.BlockSpec((HP, HP), lambda r: (0, 0)),
        ],
        out_specs=[
            pl.BlockSpec((NB, HP), lambda r: (r, 0)),
            pl.BlockSpec((NB, HP), lambda r: (r, 0)),
        ],
        out_shape=[
            jax.ShapeDtypeStruct((NP, HP), jnp.float32),
            jax.ShapeDtypeStruct((NP, HP), jnp.float32),
        ],
    )(xs, s, dinvb, b, w)


def _t5_body(x1_ref, x2_ref, x3_ref, m1_ref, m2_ref, m3_ref, fcb_ref,
             l1_ref, out_ref, acc_ref):
    r = pl.program_id(0)

    @pl.when(r == 0)
    def _():
        acc_ref[...] = jnp.zeros((8, HFC), jnp.float32)

    h = (jnp.dot(x1_ref[...], m1_ref[...], preferred_element_type=jnp.float32)
         + jnp.dot(x2_ref[...], m2_ref[...], preferred_element_type=jnp.float32)
         + jnp.dot(x3_ref[...], m3_ref[...], preferred_element_type=jnp.float32)
         + fcb_ref[0, 0])
    acc_ref[...] += lax.dot_general(h, l1_ref[...], (((0,), (0,)), ((), ())),
                                    preferred_element_type=jnp.float32)

    @pl.when(r == RB - 1)
    def _():
        out_ref[...] = acc_ref[...]


def _t5(x1, x2, x3, m1, m2, m3, fcb, lin1p):
    return pl.pallas_call(
        _t5_body,
        grid=(RB,),
        in_specs=[
            pl.BlockSpec((NB, HP), lambda r: (r, 0)),
            pl.BlockSpec((NB, HP), lambda r: (r, 0)),
            pl.BlockSpec((NB, HP), lambda r: (r, 0)),
            pl.BlockSpec((HP, 8), lambda r: (0, 0)),
            pl.BlockSpec((HP, 8), lambda r: (0, 0)),
            pl.BlockSpec((HP, 8), lambda r: (0, 0)),
            pl.BlockSpec((1, 1), lambda r: (0, 0)),
            pl.BlockSpec((NB, HFC), lambda r: (r, 0)),
        ],
        out_specs=pl.BlockSpec((8, HFC), lambda r: (0, 0)),
        out_shape=jax.ShapeDtypeStruct((8, HFC), jnp.float32),
        scratch_shapes=[pltpu.VMEM((8, HFC), jnp.float32)],
    )(x1, x2, x3, m1, m2, m3, fcb, lin1p)


def _t6_body(hp_ref, l1b_ref, l2_ref, l2b_ref, out_ref):
    z = jnp.maximum(hp_ref[0:BS, :] + l1b_ref[...], 0.0)
    logits = jnp.dot(z, l2_ref[...], preferred_element_type=jnp.float32)
    logits = logits + l2b_ref[...]
    m = jnp.max(logits, axis=-1, keepdims=True)
    s = logits - m
    out_ref[...] = s - jnp.log(jnp.sum(jnp.exp(s), axis=-1, keepdims=True))


def _t6(hp, l1b, l2, l2b):
    return pl.pallas_call(
        _t6_body,
        out_shape=jax.ShapeDtypeStruct((BS, NC), jnp.float32),
    )(hp, l1b, l2, l2b)


def _blockdiag(w, r, cdim):
    out = jnp.zeros((2 * r, 2 * cdim), w.dtype)
    out = lax.dynamic_update_slice(out, w, (0, 0))
    return lax.dynamic_update_slice(out, w, (r, cdim))


def kernel(x, batch, edge_index, W1, b1, W2, b2, W3, b3, fc_w, fc_b,
           lin1_w, lin1_b, lin2_w, lin2_b):
    xpad = jnp.pad(x, ((0, 0), (0, NP - N), (0, 0)))
    xp = jnp.concatenate([xpad[0], xpad[1]], axis=1)
    pad = jnp.full((EP - E,), NP - 1, dtype=jnp.int32)
    src2 = jnp.concatenate([edge_index[0], pad]).reshape(EPR, 128)
    dst2 = jnp.concatenate([edge_index[1], pad]).reshape(EPR, 128)

    W1b = _blockdiag(W1, F_IN, H)
    Ws = jnp.stack([_blockdiag(W2, H, H), _blockdiag(W2, H, H),
                    _blockdiag(W3, H, H), _blockdiag(W3, H, H)])
    bs = jnp.stack([jnp.concatenate([b1, b1]), jnp.concatenate([b1, b1]),
                    jnp.concatenate([b2, b2]),
                    jnp.concatenate([b3, b3])]).reshape(4, 1, HP)
    ks = jnp.arange(4)

    def step(carry, inp):
        xs, dinvb = carry
        w, b, k = inp
        s = _sc_scatter(xs, src2, dst2)

        def deg_branch(_):
            xs1, dv = _t1(xp, s, W1b)
            return xs1, dv, dv

        def layer_branch(_):
            xl, xsn = _t2(xs, s, dinvb, b, w)
            return xsn, dinvb, xl

        xsn, dvn, xl = lax.cond(k == 0, deg_branch, layer_branch, 0)
        return (xsn, dvn), xl

    ones_xs = jnp.ones((NP, HP), jnp.float32)
    _, xls = lax.scan(step, (ones_xs, ones_xs), (Ws, bs, ks))

    wt = fc_w.reshape(H, 3)
    m8 = jnp.zeros((3, HP, 8), jnp.float32)
    m8 = m8.at[:, :H, 0].set(wt.T)
    m8 = m8.at[:, H:, 1].set(wt.T)

    lin1p = jnp.pad(lin1_w, ((0, NP - N), (0, 0)))
    hp8 = _t5(xls[1], xls[2], xls[3], m8[0], m8[1], m8[2],
              fc_b.reshape(1, 1), lin1p)
    return _t6(hp8, lin1_b.reshape(1, HFC), lin2_w, lin2_b.reshape(1, NC))


# speedup vs baseline: 44.2487x; 1.1257x over previous
import functools
import jax
import jax.numpy as jnp
from jax import lax
from jax.experimental import pallas as pl
from jax.experimental.pallas import tpu as pltpu, tpu_sc as plsc

N = 15135
NP = 15360
NPH = NP // 2
E = 484320
EP = 491520
EPR = EP // 128
F_IN = 128
H = 64
HP = 128
HFC = 256
NC = 10
BS = 2

NB = 512
RB = NP // NB
CHUNK = 512
TROWS = NPH // 16

_SC_MESH = plsc.VectorSubcoreMesh(core_axis_name="c", subcore_axis_name="s")


@functools.partial(
    pl.kernel,
    out_type=jax.ShapeDtypeStruct((NP, HP), jnp.float32),
    mesh=_SC_MESH,
    scratch_types=[
        pltpu.VMEM((2, 128), jnp.int32),
        pltpu.VMEM((2, 128), jnp.int32),
        pltpu.VMEM((128,), jnp.int32),
        pltpu.VMEM((128,), jnp.int32),
        pltpu.VMEM((128,), jnp.int32),
        pltpu.VMEM((128,), jnp.int32),
        pltpu.VMEM((256, HP), jnp.float32),
        pltpu.VMEM((256, HP), jnp.float32),
        pltpu.VMEM_SHARED((NPH + 8, HP), jnp.float32),
        pltpu.SemaphoreType.DMA,
        pltpu.SemaphoreType.DMA,
    ],
)
def _sc_scatter(xs, src2, dst2c, out, sidxA, sidxB, didxA0, didxA1,
                didxB0, didxB1, rowsA, rowsB, acc, semA, semB):
    c = lax.axis_index("c")
    t = lax.axis_index("s")
    zero16 = jnp.zeros((16,), jnp.float32)

    def zfill(i, carry):
        for k in range(8):
            rowsA[i, pl.ds(k * 16, 16)] = zero16
        return carry

    lax.fori_loop(0, 240, zfill, 0)

    base = t * TROWS
    pltpu.sync_copy(rowsA.at[pl.ds(0, 240)], acc.at[pl.ds(base, 240)])
    pltpu.sync_copy(rowsA.at[pl.ds(0, 240)], acc.at[pl.ds(base + 240, 240)])
    plsc.subcore_barrier()

    sbase = t * 240
    dbase = c * EPR + t * 240

    def gather(rb, sidx, rows, sem):
        pltpu.sync_copy(src2.at[pl.ds(sbase + rb, 2)], sidx)
        pltpu.async_copy(xs.at[sidx.at[0]], rows.at[pl.ds(0, 128)], sem)
        pltpu.async_copy(xs.at[sidx.at[1]], rows.at[pl.ds(128, 128)], sem)

    def gwait(sidx, rows, sem):
        pltpu.make_async_copy(xs.at[sidx.at[0]],
                              rows.at[pl.ds(0, 128)], sem).wait()
        pltpu.make_async_copy(xs.at[sidx.at[1]],
                              rows.at[pl.ds(128, 128)], sem).wait()

    def scat(rows, d0, d1):
        pltpu.sync_copy(rows.at[pl.ds(0, 128)], acc.at[d0], add=True)
        pltpu.sync_copy(rows.at[pl.ds(128, 128)], acc.at[d1], add=True)

    gather(0, sidxA, rowsA, semA)

    def pair(i, start_next):
        rb = i * 4
        gather(rb + 2, sidxB, rowsB, semB)
        pltpu.sync_copy(dst2c.at[dbase + rb], didxA0)
        pltpu.sync_copy(dst2c.at[dbase + rb + 1], didxA1)
        gwait(sidxA, rowsA, semA)
        scat(rowsA, didxA0, didxA1)
        if start_next:
            gather(rb + 4, sidxA, rowsA, semA)
        pltpu.sync_copy(dst2c.at[dbase + rb + 2], didxB0)
        pltpu.sync_copy(dst2c.at[dbase + rb + 3], didxB1)
        gwait(sidxB, rowsB, semB)
        scat(rowsB, didxB0, didxB1)

    def body(i, carry):
        pair(i, True)
        return carry

    lax.fori_loop(0, 59, body, 0)
    pair(59, False)
    plsc.subcore_barrier()

    for q in range(2):
        sl = pl.ds(base + q * 240, 240)
        pltpu.sync_copy(acc.at[sl], rowsA.at[pl.ds(0, 240)])
        pltpu.sync_copy(rowsA.at[pl.ds(0, 240)],
                        out.at[pl.ds(c * NPH + base + q * 240, 240)])


def _t1_body(xp_ref, s_ref, w1_ref, xs1_ref, dinv_ref):
    dinvb = lax.rsqrt(s_ref[...] + 1.0)
    xw = jnp.dot(xp_ref[...], w1_ref[...], preferred_element_type=jnp.float32)
    xs1_ref[...] = dinvb * xw
    dinv_ref[...] = dinvb


def _t1(xp, s0, w1b):
    return pl.pallas_call(
        _t1_body,
        grid=(RB,),
        in_specs=[
            pl.BlockSpec((NB, 2 * F_IN), lambda r: (r, 0)),
            pl.BlockSpec((NB, HP), lambda r: (r, 0)),
            pl.BlockSpec((2 * F_IN, HP), lambda r: (0, 0)),
        ],
        out_specs=[
            pl.BlockSpec((NB, HP), lambda r: (r, 0)),
            pl.BlockSpec((NB, HP), lambda r: (r, 0)),
        ],
        out_shape=[
            jax.ShapeDtypeStruct((NP, HP), jnp.float32),
            jax.ShapeDtypeStruct((NP, HP), jnp.float32),
        ],
    )(xp, s0, w1b)


def _t2_body(xs_ref, s_ref, dinv_ref, b_ref, w_ref, xl_ref, xsn_ref):
    xl = jnp.maximum(dinv_ref[...] * (xs_ref[...] + s_ref[...]) + b_ref[...], 0.0)
    xl_ref[...] = xl
    xw = jnp.dot(xl, w_ref[...], preferred_element_type=jnp.float32)
    xsn_ref[...] = dinv_ref[...] * xw


def _t2(xs, s, dinvb, b, w):
    return pl.pallas_call(
        _t2_body,
        grid=(RB,),
        in_specs=[
            pl.BlockSpec((NB, HP), lambda r: (r, 0)),
            pl.BlockSpec((NB, HP), lambda r: (r, 0)),
            pl.BlockSpec((NB, HP), lambda r: (r, 0)),
            pl.BlockSpec((1, HP), lambda r: (0, 0)),
            pl.BlockSpec((HP, HP), lambda r: (0, 0)),
        ],
        out_specs=[
            pl.BlockSpec((NB, HP), lambda r: (r, 0)),
            pl.BlockSpec((NB, HP), lambda r: (r, 0)),
        ],
        out_shape=[
            jax.ShapeDtypeStruct((NP, HP), jnp.float32),
            jax.ShapeDtypeStruct((NP, HP), jnp.float32),
        ],
    )(xs, s, dinvb, b, w)


def _t5_body(x1_ref, x2_ref, x3_ref, m1_ref, m2_ref, m3_ref, fcb_ref,
             l1_ref, out_ref, acc_ref):
    r = pl.program_id(0)

    @pl.when(r == 0)
    def _():
        acc_ref[...] = jnp.zeros((8, HFC), jnp.float32)

    h = (jnp.dot(x1_ref[...], m1_ref[...], preferred_element_type=jnp.float32)
         + jnp.dot(x2_ref[...], m2_ref[...], preferred_element_type=jnp.float32)
         + jnp.dot(x3_ref[...], m3_ref[...], preferred_element_type=jnp.float32)
         + fcb_ref[0, 0])
    acc_ref[...] += lax.dot_general(h, l1_ref[...], (((0,), (0,)), ((), ())),
                                    preferred_element_type=jnp.float32)

    @pl.when(r == RB - 1)
    def _():
        out_ref[...] = acc_ref[...]


def _t5(x1, x2, x3, m1, m2, m3, fcb, lin1p):
    return pl.pallas_call(
        _t5_body,
        grid=(RB,),
        in_specs=[
            pl.BlockSpec((NB, HP), lambda r: (r, 0)),
            pl.BlockSpec((NB, HP), lambda r: (r, 0)),
            pl.BlockSpec((NB, HP), lambda r: (r, 0)),
            pl.BlockSpec((HP, 8), lambda r: (0, 0)),
            pl.BlockSpec((HP, 8), lambda r: (0, 0)),
            pl.BlockSpec((HP, 8), lambda r: (0, 0)),
            pl.BlockSpec((1, 1), lambda r: (0, 0)),
            pl.BlockSpec((NB, HFC), lambda r: (r, 0)),
        ],
        out_specs=pl.BlockSpec((8, HFC), lambda r: (0, 0)),
        out_shape=jax.ShapeDtypeStruct((8, HFC), jnp.float32),
        scratch_shapes=[pltpu.VMEM((8, HFC), jnp.float32)],
    )(x1, x2, x3, m1, m2, m3, fcb, lin1p)


def _t6_body(hp_ref, l1b_ref, l2_ref, l2b_ref, out_ref):
    z = jnp.maximum(hp_ref[0:BS, :] + l1b_ref[...], 0.0)
    logits = jnp.dot(z, l2_ref[...], preferred_element_type=jnp.float32)
    logits = logits + l2b_ref[...]
    m = jnp.max(logits, axis=-1, keepdims=True)
    s = logits - m
    out_ref[...] = s - jnp.log(jnp.sum(jnp.exp(s), axis=-1, keepdims=True))


def _t6(hp, l1b, l2, l2b):
    return pl.pallas_call(
        _t6_body,
        out_shape=jax.ShapeDtypeStruct((BS, NC), jnp.float32),
    )(hp, l1b, l2, l2b)


def _blockdiag(w, r, cdim):
    out = jnp.zeros((2 * r, 2 * cdim), w.dtype)
    out = lax.dynamic_update_slice(out, w, (0, 0))
    return lax.dynamic_update_slice(out, w, (r, cdim))


def kernel(x, batch, edge_index, W1, b1, W2, b2, W3, b3, fc_w, fc_b,
           lin1_w, lin1_b, lin2_w, lin2_b):
    xpad = jnp.pad(x, ((0, 0), (0, NP - N), (0, 0)))
    xp = jnp.concatenate([xpad[0], xpad[1]], axis=1)
    pad = jnp.full((EP - E,), NP - 1, dtype=jnp.int32)
    src2 = jnp.concatenate([edge_index[0], pad]).reshape(EPR, 128)
    dst = jnp.concatenate([edge_index[1], pad])
    d0 = jnp.where(dst < NPH, dst, NPH)
    d1m = dst - NPH
    d1 = jnp.where(d1m >= 0, d1m, NPH)
    dst2c = jnp.concatenate([d0, d1]).reshape(2 * EPR, 128)

    W1b = _blockdiag(W1, F_IN, H)
    Ws = jnp.stack([_blockdiag(W2, H, H), _blockdiag(W2, H, H),
                    _blockdiag(W3, H, H), _blockdiag(W3, H, H)])
    bs = jnp.stack([jnp.concatenate([b1, b1]), jnp.concatenate([b1, b1]),
                    jnp.concatenate([b2, b2]),
                    jnp.concatenate([b3, b3])]).reshape(4, 1, HP)
    ks = jnp.arange(4)

    def step(carry, inp):
        xs, dinvb = carry
        w, b, k = inp
        s = _sc_scatter(xs, src2, dst2c)

        def deg_branch(_):
            xs1, dv = _t1(xp, s, W1b)
            return xs1, dv, dv

        def layer_branch(_):
            xl, xsn = _t2(xs, s, dinvb, b, w)
            return xsn, dinvb, xl

        xsn, dvn, xl = lax.cond(k == 0, deg_branch, layer_branch, 0)
        return (xsn, dvn), xl

    ones_xs = jnp.ones((NP, HP), jnp.float32)
    _, xls = lax.scan(step, (ones_xs, ones_xs), (Ws, bs, ks))

    wt = fc_w.reshape(H, 3)
    m8 = jnp.zeros((3, HP, 8), jnp.float32)
    m8 = m8.at[:, :H, 0].set(wt.T)
    m8 = m8.at[:, H:, 1].set(wt.T)

    lin1p = jnp.pad(lin1_w, ((0, NP - N), (0, 0)))
    hp8 = _t5(xls[1], xls[2], xls[3], m8[0], m8[1], m8[2],
              fc_b.reshape(1, 1), lin1p)
    return _t6(hp8, lin1_b.reshape(1, HFC), lin2_w, lin2_b.reshape(1, NC))
